# Initial kernel scaffold; baseline (speedup 1.0000x reference)
import jax
import jax.numpy as jnp
from jax.experimental import pallas as pl


def kernel(x, y):
    raise NotImplementedError("write your pallas kernel here")


# speedup vs baseline: 12.9268x; 12.9268x over previous
import jax
import jax.numpy as jnp
from jax.experimental import pallas as pl
from jax.experimental.pallas import tpu as pltpu

_K = 16
_RB = 256


def _knn_block_kernel(x_ref, y_ref, out_ref):
    x = x_ref[0]
    y = y_ref[0]
    n = y.shape[-1]
    xn = x / jnp.maximum(jnp.sqrt(jnp.sum(x * x, axis=0, keepdims=True)), 1e-12)
    yn = y / jnp.maximum(jnp.sqrt(jnp.sum(y * y, axis=0, keepdims=True)), 1e-12)
    x2 = jnp.sum(xn * xn, axis=0)
    y2 = jnp.sum(yn * yn, axis=0)
    inner = jax.lax.dot_general(
        xn.astype(jnp.bfloat16), yn.astype(jnp.bfloat16),
        (((0,), (0,)), ((), ())),
        preferred_element_type=jnp.float32,
    )
    dist = (x2[:, None] + (-2.0) * inner) + y2[None, :]

    iota = jax.lax.broadcasted_iota(jnp.int32, dist.shape, 1)
    cur = dist
    cols = []
    for _ in range(_K):
        m = jnp.min(cur, axis=1, keepdims=True)
        am = jnp.min(jnp.where(cur == m, iota, n), axis=1)
        cols.append(am)
        cur = jnp.where(iota == am[:, None], jnp.inf, cur)
    out_ref[0] = jnp.stack(cols, axis=1)


def kernel(x, y):
    b, c, n, _ = x.shape
    xs = x[..., 0]
    ys = y[..., 0]
    nn_idx = pl.pallas_call(
        _knn_block_kernel,
        grid=(b, n // _RB),
        in_specs=[
            pl.BlockSpec((1, c, _RB), lambda bi, i: (bi, 0, i)),
            pl.BlockSpec((1, c, n), lambda bi, i: (bi, 0, 0)),
        ],
        out_specs=pl.BlockSpec((1, _RB, _K), lambda bi, i: (bi, i, 0)),
        out_shape=jax.ShapeDtypeStruct((b, n, _K), jnp.int32),
    )(xs, ys)
    center_idx = jnp.broadcast_to(
        jnp.arange(n, dtype=jnp.int32)[None, :, None], (b, n, _K)
    )
    return jnp.stack((nn_idx, center_idx), axis=0)
